# baseline (speedup 1.0000x reference)
import functools
import math

import jax
import jax.numpy as jnp
from jax import lax
from jax.experimental import pallas as pl
from jax.experimental.pallas import tpu as pltpu
from jax.experimental.pallas import tpu_sc as plsc

_INV_TEMP = 1.0 / 0.7
_INV_SQRT2 = 1.0 / math.sqrt(2.0)
_BLK = 2048


def _mlp_body(x_ref, w1t_ref, b1_ref, w2t_ref, b2_ref, out_ref):
    x = x_ref[...]
    h = (lax.dot_general(x, w1t_ref[...], (((1,), (1,)), ((), ())),
                         preferred_element_type=jnp.float32)
         + b1_ref[...].reshape(1, -1))
    h = 0.5 * h * (1.0 + lax.erf(h * _INV_SQRT2))
    logits = (lax.dot_general(h, w2t_ref[...], (((1,), (1,)), ((), ())),
                              preferred_element_type=jnp.float32)
              + b2_ref[...].reshape(1, -1))
    out_ref[...] = (logits * _INV_TEMP).T


def _mlp_logits_t(x, W1t, b1, W2t, b2):
    n_tokens, in_dim = x.shape
    hidden = W1t.shape[0]
    n_exp = W2t.shape[0]
    blk = min(_BLK, n_tokens)
    return pl.pallas_call(
        _mlp_body,
        grid=(n_tokens // blk,),
        in_specs=[
            pl.BlockSpec((blk, in_dim), lambda i: (i, 0)),
            pl.BlockSpec((hidden, in_dim), lambda i: (0, 0)),
            pl.BlockSpec((hidden,), lambda i: (0,)),
            pl.BlockSpec((n_exp, hidden), lambda i: (0, 0)),
            pl.BlockSpec((n_exp,), lambda i: (0,)),
        ],
        out_specs=pl.BlockSpec((n_exp, blk), lambda i: (0, i)),
        out_shape=jax.ShapeDtypeStruct((n_exp, n_tokens), jnp.float32),
        compiler_params=pltpu.CompilerParams(
            dimension_semantics=("arbitrary",),
        ),
    )(x, W1t, b1, W2t, b2)


def _tree(op, vs):
    while len(vs) > 1:
        vs = [op(vs[i], vs[i + 1]) if i + 1 < len(vs) else vs[i]
              for i in range(0, len(vs), 2)]
    return vs[0]


def _make_router(n_tokens, n_exp):
    info = plsc.get_sparse_core_info()
    nw = info.num_cores * info.num_subcores
    lanes = info.num_lanes
    c = n_tokens // nw

    @functools.partial(
        pl.kernel,
        mesh=plsc.VectorSubcoreMesh(core_axis_name="c", subcore_axis_name="s"),
        out_type=jax.ShapeDtypeStruct((n_exp, n_tokens), jnp.float32),
        scratch_types=[
            pltpu.VMEM((n_exp, c), jnp.float32),
            pltpu.VMEM((n_exp, c), jnp.float32),
        ],
    )
    def router(s_hbm, out_hbm, s_v, o_v):
        wid = lax.axis_index("s") * info.num_cores + lax.axis_index("c")
        base = wid * c
        pltpu.sync_copy(s_hbm.at[:, pl.ds(base, c)], s_v)
        big = jnp.full((lanes,), n_exp, jnp.int32)

        def group(g, carry):
            t0 = g * lanes
            s = [s_v[j, pl.ds(t0, lanes)] for j in range(n_exp)]
            m = _tree(jnp.maximum, s)
            e = [jnp.exp(v - m) for v in s]
            ssum = _tree(jnp.add, e)
            gates = [v / ssum for v in e]
            m1 = _tree(jnp.maximum, gates)
            i1 = _tree(jnp.minimum,
                       [jnp.where(gates[j] == m1, j, big)
                        for j in range(n_exp)])
            g2 = [jnp.where(i1 == j, -1.0, gates[j]) for j in range(n_exp)]
            m2 = _tree(jnp.maximum, g2)
            i2 = _tree(jnp.minimum,
                       [jnp.where(g2[j] == m2, j, big)
                        for j in range(n_exp)])
            denom = m1 + m2 + 1e-8
            for j in range(n_exp):
                sel = (i1 == j) | (i2 == j)
                o_v[j, pl.ds(t0, lanes)] = (
                    jnp.where(sel, gates[j], 0.0) / denom)
            return carry

        lax.fori_loop(0, c // lanes, group, 0)
        pltpu.sync_copy(o_v, out_hbm.at[:, pl.ds(base, c)])

    return router


@jax.jit
def kernel(instrument_logits, W1, b1, W2, b2):
    n_tokens = instrument_logits.shape[0]
    n_exp = W2.shape[1]
    s_t = _mlp_logits_t(instrument_logits, W1.T, b1, W2.T, b2)
    router = _make_router(n_tokens, n_exp)
    return router(s_t).T


# speedup vs baseline: 1.6851x; 1.6851x over previous
import functools
import math

import jax
import jax.numpy as jnp
from jax.experimental import pallas as pl
from jax.experimental.pallas import tpu as pltpu

_NUM_EXPERTS = 16
_TOP_K = 2
_INV_TEMP = 1.0 / 0.7
_INV_SQRT2 = 1.0 / math.sqrt(2.0)
_BLK = 2048


def _router_body(x_ref, w1t_ref, b1_ref, w2t_ref, b2_ref, out_ref):
    x = x_ref[...]
    w1t = w1t_ref[...]
    w2t = w2t_ref[...]
    h = (jax.lax.dot_general(
            x, w1t, (((1,), (1,)), ((), ())),
            preferred_element_type=jnp.float32)
         + b1_ref[...].reshape(1, -1))
    h = 0.5 * h * (1.0 + jax.lax.erf(h * _INV_SQRT2))
    logits = (jax.lax.dot_general(
                  h, w2t, (((1,), (1,)), ((), ())),
                  preferred_element_type=jnp.float32)
              + b2_ref[...].reshape(1, -1))
    s = logits * _INV_TEMP
    s = s - jnp.max(s, axis=-1, keepdims=True)
    e = jnp.exp(s)
    gates = e / jnp.sum(e, axis=-1, keepdims=True)

    n_exp = gates.shape[-1]
    row_i = jax.lax.broadcasted_iota(jnp.int32, (n_exp, n_exp), 0)
    col_i = jax.lax.broadcasted_iota(jnp.int32, (n_exp, n_exp), 1)
    ut = (row_i < col_i).astype(jnp.float32)
    m1 = jnp.max(gates, axis=-1, keepdims=True)
    is1 = gates == m1
    pre1 = jnp.dot(is1.astype(jnp.float32), ut,
                   preferred_element_type=jnp.float32)
    occ1 = is1 & (pre1 == 0.0)
    g2 = jnp.where(occ1, -1.0, gates)
    m2 = jnp.max(g2, axis=-1, keepdims=True)
    is2 = g2 == m2
    pre2 = jnp.dot(is2.astype(jnp.float32), ut,
                   preferred_element_type=jnp.float32)
    occ2 = is2 & (pre2 == 0.0)
    mask = occ1 | occ2

    gg = jnp.where(mask, gates, 0.0)
    out_ref[...] = (gg / (m1 + m2 + 1e-8)).T


@functools.partial(jax.jit, static_argnames=())
def kernel(instrument_logits, W1, b1, W2, b2):
    n_tokens, in_dim = instrument_logits.shape
    hidden = W1.shape[1]
    n_exp = W2.shape[1]
    blk = min(_BLK, n_tokens)
    grid = (n_tokens // blk,)
    return pl.pallas_call(
        _router_body,
        grid=grid,
        in_specs=[
            pl.BlockSpec((blk, in_dim), lambda i: (i, 0)),
            pl.BlockSpec((hidden, in_dim), lambda i: (0, 0)),
            pl.BlockSpec((hidden,), lambda i: (0,)),
            pl.BlockSpec((n_exp, hidden), lambda i: (0, 0)),
            pl.BlockSpec((n_exp,), lambda i: (0,)),
        ],
        out_specs=pl.BlockSpec((n_exp, blk), lambda i: (0, i)),
        out_shape=jax.ShapeDtypeStruct((n_exp, n_tokens), jnp.float32),
        compiler_params=pltpu.CompilerParams(
            dimension_semantics=("parallel",),
        ),
    )(instrument_logits, W1.T, b1, W2.T, b2).T
